# baseline (speedup 1.0000x reference)
import functools

import numpy as np
import jax
import jax.numpy as jnp
from jax import lax
from jax.experimental import pallas as pl
from jax.experimental.pallas import tpu as pltpu
from jax.experimental.pallas import tpu_sc as plsc

_LANES = 16
_CHUNK = 128


def _cos_sin_table(seq, d):
    j = np.arange(d)
    angle = (1.0 / np.power(10000.0, 2.0 * (j // 2) / d)).astype(np.float32)
    pos = np.arange(seq, dtype=np.float32)[:, None]
    phase = (pos * angle[None, :]).astype(np.float64)
    return jnp.asarray(
        np.stack([np.cos(phase), np.sin(phase)]).astype(np.float32))


@functools.lru_cache(maxsize=None)
def _build_sc_kernel(seq, n_chunks, d):
    info = plsc.get_sparse_core_info()
    n_workers = info.num_cores * info.num_subcores
    n_tasks = seq * n_chunks
    per_w = n_tasks // n_workers
    assert per_w * n_workers == n_tasks and per_w % 2 == 0
    n_groups = d // _LANES
    batch = n_chunks * _CHUNK
    mesh = plsc.VectorSubcoreMesh(core_axis_name="c", subcore_axis_name="s")

    @functools.partial(
        pl.kernel,
        mesh=mesh,
        out_type=jax.ShapeDtypeStruct((2, batch, seq, d), jnp.float32),
        scratch_types=[
            pltpu.VMEM((_CHUNK,), jnp.int32), pltpu.VMEM((_CHUNK,), jnp.int32),
            pltpu.VMEM((d,), jnp.float32), pltpu.VMEM((d,), jnp.float32),
            pltpu.VMEM((d,), jnp.float32), pltpu.VMEM((d,), jnp.float32),
            pltpu.VMEM((_CHUNK, d), jnp.float32), pltpu.VMEM((_CHUNK, d), jnp.float32),
            pltpu.VMEM((_CHUNK, d), jnp.float32), pltpu.VMEM((_CHUNK, d), jnp.float32),
            pltpu.VMEM((_CHUNK, d), jnp.float32), pltpu.VMEM((_CHUNK, d), jnp.float32),
            pltpu.SemaphoreType.DMA, pltpu.SemaphoreType.DMA,
            pltpu.SemaphoreType.DMA, pltpu.SemaphoreType.DMA,
            pltpu.SemaphoreType.DMA, pltpu.SemaphoreType.DMA,
        ],
    )
    def k(xt_hbm, w_hbm, cs_hbm, out_hbm,
          idx0, idx1, cos0, cos1, sin0, sin1,
          rows0, rows1, real0, real1, ph0, ph1,
          sg0, sg1, si0, si1, so0, so1):
        wid = lax.axis_index("s") * info.num_cores + lax.axis_index("c")
        base = wid * per_w

        def src_pos_chunk(t):
            g = base + t
            return g // n_chunks, g % n_chunks

        def input_copies(t, idx_v, cos_v, sin_v, si):
            pos, ch = src_pos_chunk(t)
            return (
                pltpu.make_async_copy(xt_hbm.at[pos, ch], idx_v, si),
                pltpu.make_async_copy(cs_hbm.at[0, pos], cos_v, si),
                pltpu.make_async_copy(cs_hbm.at[1, pos], sin_v, si),
            )

        def out_copies(t, real_v, ph_v, so):
            pos, ch = src_pos_chunk(t)
            sl = pl.ds(ch * _CHUNK, _CHUNK)
            return (
                pltpu.make_async_copy(real_v, out_hbm.at[0, sl, pos], so),
                pltpu.make_async_copy(ph_v, out_hbm.at[1, sl, pos], so),
            )

        def do_task(t, idx_a, cos_a, sin_a, rows_a, real_a, ph_a, sg_a, so_a,
                    idx_b, cos_b, sin_b, rows_b, sg_b, si_b):
            @pl.when(t + 1 < per_w)
            def _():
                for c in input_copies(t + 1, idx_b, cos_b, sin_b, si_b):
                    c.start()

            pltpu.make_async_copy(w_hbm.at[idx_a], rows_a, sg_a).wait()

            @pl.when(t + 1 < per_w)
            def _():
                for c in input_copies(t + 1, idx_b, cos_b, sin_b, si_b):
                    c.wait()
                pltpu.make_async_copy(w_hbm.at[idx_b], rows_b, sg_b).start()

            @pl.when(t >= 2)
            def _():
                for c in out_copies(t - 2, real_a, ph_a, so_a):
                    c.wait()

            cvec = [cos_a[pl.ds(g * _LANES, _LANES)] for g in range(n_groups)]
            svec = [sin_a[pl.ds(g * _LANES, _LANES)] for g in range(n_groups)]

            @plsc.parallel_loop(0, _CHUNK, 1, unroll=4)
            def _row(r):
                for g in range(n_groups):
                    sl = pl.ds(g * _LANES, _LANES)
                    v = rows_a[r, sl]
                    real_a[r, sl] = v * cvec[g]
                    ph_a[r, sl] = v * svec[g]

            for c in out_copies(t, real_a, ph_a, so_a):
                c.start()

        for c in input_copies(0, idx0, cos0, sin0, si0):
            c.start()
        for c in input_copies(0, idx0, cos0, sin0, si0):
            c.wait()
        pltpu.make_async_copy(w_hbm.at[idx0], rows0, sg0).start()

        def pair(i, c):
            t0 = 2 * i
            do_task(t0, idx0, cos0, sin0, rows0, real0, ph0, sg0, so0,
                    idx1, cos1, sin1, rows1, sg1, si1)
            do_task(t0 + 1, idx1, cos1, sin1, rows1, real1, ph1, sg1, so1,
                    idx0, cos0, sin0, rows0, sg0, si0)
            return c

        lax.fori_loop(0, per_w // 2, pair, 0)

        for c in out_copies(per_w - 2, real0, ph0, so0):
            c.wait()
        for c in out_copies(per_w - 1, real1, ph1, so1):
            c.wait()

    return k


def kernel(x, W):
    batch, seq = x.shape
    d = W.shape[1]
    xt = x.T.reshape(seq, batch // _CHUNK, _CHUNK)
    cs = _cos_sin_table(seq, d)
    return _build_sc_kernel(seq, batch // _CHUNK, d)(xt, W, cs)


# speedup vs baseline: 1.0057x; 1.0057x over previous
import functools

import numpy as np
import jax
import jax.numpy as jnp
from jax import lax
from jax.experimental import pallas as pl
from jax.experimental.pallas import tpu as pltpu
from jax.experimental.pallas import tpu_sc as plsc

_LANES = 16
_CHUNK = 128


def _cos_sin_table(seq, d):
    j = np.arange(d)
    angle = (1.0 / np.power(10000.0, 2.0 * (j // 2) / d)).astype(np.float32)
    pos = np.arange(seq, dtype=np.float32)[:, None]
    phase = (pos * angle[None, :]).astype(np.float64)
    return jnp.asarray(
        np.stack([np.cos(phase), np.sin(phase)]).astype(np.float32))


@functools.lru_cache(maxsize=None)
def _build_sc_kernel(seq, n_chunks, d):
    info = plsc.get_sparse_core_info()
    n_workers = info.num_cores * info.num_subcores
    n_tasks = seq * n_chunks
    per_w = n_tasks // n_workers
    assert per_w * n_workers == n_tasks and per_w % 2 == 0
    n_groups = d // _LANES
    batch = n_chunks * _CHUNK
    mesh = plsc.VectorSubcoreMesh(core_axis_name="c", subcore_axis_name="s")

    @functools.partial(
        pl.kernel,
        mesh=mesh,
        out_type=jax.ShapeDtypeStruct((2, batch, seq, d), jnp.float32),
        scratch_types=[
            pltpu.VMEM((_CHUNK,), jnp.int32), pltpu.VMEM((_CHUNK,), jnp.int32),
            pltpu.VMEM((d,), jnp.float32), pltpu.VMEM((d,), jnp.float32),
            pltpu.VMEM((d,), jnp.float32), pltpu.VMEM((d,), jnp.float32),
            pltpu.VMEM((_CHUNK, d), jnp.float32), pltpu.VMEM((_CHUNK, d), jnp.float32),
            pltpu.VMEM((_CHUNK, d), jnp.float32), pltpu.VMEM((_CHUNK, d), jnp.float32),
            pltpu.VMEM((_CHUNK, d), jnp.float32), pltpu.VMEM((_CHUNK, d), jnp.float32),
            pltpu.SemaphoreType.DMA, pltpu.SemaphoreType.DMA,
            pltpu.SemaphoreType.DMA, pltpu.SemaphoreType.DMA,
            pltpu.SemaphoreType.DMA, pltpu.SemaphoreType.DMA,
        ],
    )
    def k(xt_hbm, w_hbm, cs_hbm, out_hbm,
          idx0, idx1, cos0, cos1, sin0, sin1,
          rows0, rows1, real0, real1, ph0, ph1,
          sg0, sg1, si0, si1, so0, so1):
        wid = lax.axis_index("s") * info.num_cores + lax.axis_index("c")
        base = wid * per_w

        def src_pos_chunk(t):
            g = base + t
            return g // n_chunks, g % n_chunks

        def input_copies(t, idx_v, cos_v, sin_v, si):
            pos, ch = src_pos_chunk(t)
            return (
                pltpu.make_async_copy(xt_hbm.at[pos, ch], idx_v, si),
                pltpu.make_async_copy(cs_hbm.at[0, pos], cos_v, si),
                pltpu.make_async_copy(cs_hbm.at[1, pos], sin_v, si),
            )

        def out_copies(t, real_v, ph_v, so):
            pos, ch = src_pos_chunk(t)
            sl = pl.ds(ch * _CHUNK, _CHUNK)
            return (
                pltpu.make_async_copy(real_v, out_hbm.at[0, sl, pos], so),
                pltpu.make_async_copy(ph_v, out_hbm.at[1, sl, pos], so),
            )

        def do_task(t, idx_a, cos_a, sin_a, rows_a, real_a, ph_a, sg_a, so_a, si_a,
                    idx_b, cos_b, sin_b, rows_b, sg_b, si_b):
            @pl.when(t + 1 < per_w)
            def _():
                for c in input_copies(t + 1, idx_b, cos_b, sin_b, si_b):
                    c.wait()
                pltpu.make_async_copy(w_hbm.at[idx_b], rows_b, sg_b).start()

            pltpu.make_async_copy(w_hbm.at[idx_a], rows_a, sg_a).wait()

            cvec = [cos_a[pl.ds(g * _LANES, _LANES)] for g in range(n_groups)]
            svec = [sin_a[pl.ds(g * _LANES, _LANES)] for g in range(n_groups)]

            @pl.when(t + 2 < per_w)
            def _():
                for c in input_copies(t + 2, idx_a, cos_a, sin_a, si_a):
                    c.start()

            @pl.when(t >= 2)
            def _():
                for c in out_copies(t - 2, real_a, ph_a, so_a):
                    c.wait()

            @plsc.parallel_loop(0, _CHUNK, 1, unroll=4)
            def _row(r):
                for g in range(n_groups):
                    sl = pl.ds(g * _LANES, _LANES)
                    v = rows_a[r, sl]
                    real_a[r, sl] = v * cvec[g]
                    ph_a[r, sl] = v * svec[g]

            for c in out_copies(t, real_a, ph_a, so_a):
                c.start()

        for c in input_copies(0, idx0, cos0, sin0, si0):
            c.start()
        for c in input_copies(1, idx1, cos1, sin1, si1):
            c.start()
        for c in input_copies(0, idx0, cos0, sin0, si0):
            c.wait()
        pltpu.make_async_copy(w_hbm.at[idx0], rows0, sg0).start()

        def pair(i, c):
            t0 = 2 * i
            do_task(t0, idx0, cos0, sin0, rows0, real0, ph0, sg0, so0, si0,
                    idx1, cos1, sin1, rows1, sg1, si1)
            do_task(t0 + 1, idx1, cos1, sin1, rows1, real1, ph1, sg1, so1, si1,
                    idx0, cos0, sin0, rows0, sg0, si0)
            return c

        lax.fori_loop(0, per_w // 2, pair, 0)

        for c in out_copies(per_w - 2, real0, ph0, so0):
            c.wait()
        for c in out_copies(per_w - 1, real1, ph1, so1):
            c.wait()

    return k


def kernel(x, W):
    batch, seq = x.shape
    d = W.shape[1]
    xt = x.T.reshape(seq, batch // _CHUNK, _CHUNK)
    cs = _cos_sin_table(seq, d)
    return _build_sc_kernel(seq, batch // _CHUNK, d)(xt, W, cs)
